# baseline (speedup 1.0000x reference)
import jax
import jax.numpy as jnp
from jax.experimental import pallas as pl

NC = 1000
VAL = 1001.0
ROWS_PER_BLOCK = 4096


def _smooth_onehot_block(lab_ref, out_ref):
    lab = lab_ref[...]
    col = jax.lax.broadcasted_iota(jnp.int32, (lab.shape[0], NC), 1)
    out_ref[...] = jnp.where(lab == col, VAL, 1.0)


def kernel(labels):
    n = labels.shape[0]
    r = ROWS_PER_BLOCK
    lab2d = labels.astype(jnp.int32).reshape(n, 1)
    return pl.pallas_call(
        _smooth_onehot_block,
        grid=(n // r,),
        in_specs=[pl.BlockSpec((r, 1), lambda i: (i, 0))],
        out_specs=pl.BlockSpec((r, NC), lambda i: (i, 0)),
        out_shape=jax.ShapeDtypeStruct((n, NC), jnp.float32),
    )(lab2d)
